# baseline (speedup 1.0000x reference)
import jax
import jax.numpy as jnp
from jax import lax
from jax.experimental import pallas as pl
from jax.experimental.pallas import tpu as pltpu
from jax.experimental.pallas import tpu_sc as plsc

NUM_EMB = 1000000
DIM = 64
BATCH = 16384
HIST = 50

NC = 2
NS = 16
NW = NC * NS

B = BATCH * HIST

KCOLS_FULL = NUM_EMB // 128
K_PER_W = KCOLS_FULL // NW
K_REM = KCOLS_FULL - K_PER_W * NW
TAIL_ROWS = NUM_EMB - KCOLS_FULL * 128

B_PER_W = BATCH // NW
NB = 4
CHUNK = NB * HIST
CWIN = 256
N_CHUNKS = B_PER_W // NB
assert CHUNK % 8 == 0 and N_CHUNKS % 2 == 0

_IOTA16 = None


def _transpose_body(wt_hbm, wtail_hbm, w2x_hbm, tiles_v, outb_v, tail_v,
                    l_sems, s_sems):
    wid = lax.axis_index("s") * NC + lax.axis_index("c")
    base_k = wid * K_PER_W

    iota16 = lax.broadcasted_iota(jnp.int32, (16,), 0)

    def transpose_tile(tv, ov, nrows, l0=0):
        for r in range(nrows):
            lvec = jnp.full((16,), l0 + r, jnp.int32)
            for g in range(4):
                vals = plsc.load_gather(tv, [iota16 + 16 * g, lvec])
                ov[r, pl.ds(16 * g, 16)] = vals

    def start_load(p, k):
        off = pl.multiple_of(k * 128, 128)
        pltpu.async_copy(wt_hbm.at[:, pl.ds(off, 128)], tiles_v.at[p], l_sems[p])

    def wait_load(p):
        pltpu.make_async_copy(
            wt_hbm.at[:, pl.ds(0, 128)], tiles_v.at[p], l_sems[p]
        ).wait()

    def start_store(p, k):
        off = pl.multiple_of(k * 128, 128)
        pltpu.async_copy(outb_v.at[p], w2x_hbm.at[pl.ds(off, 128)], s_sems[p])

    def wait_store(p):
        pltpu.make_async_copy(
            outb_v.at[p], w2x_hbm.at[pl.ds(0, 128)], s_sems[p]
        ).wait()

    start_load(0, base_k)
    start_load(1, base_k + 1)

    @pl.loop(0, K_PER_W, step=2)
    def _(t):
        for p in range(2):
            k = base_k + t + p
            wait_load(p)

            @pl.when(t + p >= 2)
            def _():
                wait_store(p)

            transpose_tile(tiles_v.at[p], outb_v.at[p], 128)
            start_store(p, k)
            nxt = jnp.minimum(k + 2, base_k + K_PER_W - 1)
            start_load(p, nxt)

    for p in range(2):
        wait_load(p)
        wait_store(p)

    @pl.when(wid < K_REM)
    def _():
        k = KCOLS_FULL - K_REM + wid
        off = k * 128
        pltpu.sync_copy(wt_hbm.at[:, pl.ds(off, 128)], tiles_v.at[0])
        transpose_tile(tiles_v.at[0], outb_v.at[0], 128)
        pltpu.sync_copy(outb_v.at[0], w2x_hbm.at[pl.ds(off, 128)])

    @pl.when(wid == K_REM)
    def _():
        pltpu.sync_copy(wtail_hbm, tail_v)
        for r in range(TAIL_ROWS):
            for g in range(4):
                outb_v[1, r, pl.ds(16 * g, 16)] = tail_v[r, pl.ds(16 * g, 16)]
        pltpu.sync_copy(
            outb_v.at[1, pl.ds(0, TAIL_ROWS)],
            w2x_hbm.at[pl.ds(NUM_EMB - TAIL_ROWS, TAIL_ROWS)],
        )


def _gather_body(idx_hbm, w2x_hbm, out_hbm, idxb0, idxb1, g0, g1,
                 slab0, slab1, i_sems, g_sems, s_sems):
    idxb = (idxb0, idxb1)
    gbuf = (g0, g1)
    slab = (slab0, slab1)
    wid = lax.axis_index("s") * NC + lax.axis_index("c")
    base_b = wid * B_PER_W
    base_l = base_b * HIST

    def start_idx(p, c):
        off = pl.multiple_of(base_l + c * CHUNK, 8)
        pltpu.async_copy(idx_hbm.at[pl.ds(off, CWIN)], idxb[p], i_sems[p])

    def wait_idx(p):
        pltpu.make_async_copy(
            idx_hbm.at[pl.ds(0, CWIN)], idxb[p], i_sems[p]
        ).wait()

    def start_gather(p):
        pltpu.async_copy(w2x_hbm.at[idxb[p]], gbuf[p], g_sems[p])

    def wait_gather(p):
        pltpu.make_async_copy(
            w2x_hbm.at[idxb[p]], gbuf[p], g_sems[p]
        ).wait()

    def start_store(p, c):
        b0 = pl.multiple_of(base_b + c * NB, NB)
        pltpu.async_copy(slab[p], out_hbm.at[pl.ds(b0, NB)], s_sems[p])

    def wait_store(p):
        pltpu.make_async_copy(
            slab[p], out_hbm.at[pl.ds(0, NB)], s_sems[p]
        ).wait()

    def extract(p):
        for j in range(CHUNK):
            b_l, h = j // HIST, j % HIST
            for g in range(4):
                slab[p][b_l, h, pl.ds(16 * g, 16)] = gbuf[p][j, pl.ds(16 * g, 16)]

    start_idx(0, 0)
    start_idx(1, 1)
    wait_idx(0)
    start_gather(0)

    @pl.loop(0, N_CHUNKS, step=2)
    def _(c0):
        for p in range(2):
            c = c0 + p
            wait_gather(p)
            start_idx(p, jnp.minimum(c + 2, N_CHUNKS - 1))
            wait_idx(1 - p)
            start_gather(1 - p)

            @pl.when(c >= 2)
            def _():
                wait_store(p)

            extract(p)
            start_store(p, c)

    wait_gather(0)
    wait_idx(1)
    for p in range(2):
        wait_store(p)


@jax.jit
def _embed(token_ids, W):
    idx = jnp.pad(token_ids.reshape(B), (0, CWIN - CHUNK))
    wt = W.T
    mesh = plsc.VectorSubcoreMesh(core_axis_name="c", subcore_axis_name="s")

    wtail = W[NUM_EMB - TAIL_ROWS:, :]
    w2x = pl.kernel(
        _transpose_body,
        out_type=jax.ShapeDtypeStruct((NUM_EMB, 128), jnp.float32),
        mesh=mesh,
        scratch_types=[
            pltpu.VMEM((2, 64, 128), jnp.float32),
            pltpu.VMEM((2, 128, 128), jnp.float32),
            pltpu.VMEM((TAIL_ROWS, DIM), jnp.float32),
            [pltpu.SemaphoreType.DMA] * 2,
            [pltpu.SemaphoreType.DMA] * 2,
        ],
        compiler_params=pltpu.CompilerParams(needs_layout_passes=False),
    )(wt, wtail)

    out = pl.kernel(
        _gather_body,
        out_type=jax.ShapeDtypeStruct((BATCH, HIST, DIM), jnp.float32),
        mesh=mesh,
        scratch_types=[
            pltpu.VMEM((CWIN,), jnp.int32),
            pltpu.VMEM((CWIN,), jnp.int32),
            pltpu.VMEM((CWIN, 128), jnp.float32),
            pltpu.VMEM((CWIN, 128), jnp.float32),
            pltpu.VMEM((NB, HIST, DIM), jnp.float32),
            pltpu.VMEM((NB, HIST, DIM), jnp.float32),
            [pltpu.SemaphoreType.DMA] * 2,
            [pltpu.SemaphoreType.DMA] * 2,
            [pltpu.SemaphoreType.DMA] * 2,
        ],
    )(idx, w2x)
    return out


def kernel(token_ids, W):
    return _embed(token_ids, W)


# speedup vs baseline: 1.9060x; 1.9060x over previous
import jax
import jax.numpy as jnp
from jax import lax
from jax.experimental import pallas as pl
from jax.experimental.pallas import tpu as pltpu
from jax.experimental.pallas import tpu_sc as plsc

NUM_EMB = 1000000
DIM = 64
BATCH = 16384
HIST = 50

NC = 2
NS = 16
NW = NC * NS

B = BATCH * HIST

KCOLS_FULL = NUM_EMB // 128
K_PER_W = KCOLS_FULL // NW
K_REM = KCOLS_FULL - K_PER_W * NW
TAIL_ROWS = NUM_EMB - KCOLS_FULL * 128

B_PER_W = BATCH // NW
NB = 4
CHUNK = NB * HIST
CWIN = 256
N_CHUNKS = B_PER_W // NB
assert CHUNK % 8 == 0 and N_CHUNKS % 2 == 0

_IOTA16 = None


def _transpose_body(wt_hbm, wtail_hbm, w2x_hbm, tiles_v, outb_v, tail_v,
                    l_sems, s_sems):
    wid = lax.axis_index("s") * NC + lax.axis_index("c")
    base_k = wid * K_PER_W

    iota16 = lax.broadcasted_iota(jnp.int32, (16,), 0)

    perms = [(iota16 + j) & 15 for j in range(16)]

    def transpose_tile(tv, ov):
        @pl.loop(0, 32)
        def _(t):
            d0 = (t // 8) * 16
            l0 = (t % 8) * 16
            dvec = iota16 + d0
            for j in range(16):
                lvec = perms[j] + l0
                vals = plsc.load_gather(tv, [dvec, lvec])
                plsc.store_scatter(ov, [lvec, dvec], vals)

    def start_load(p, k):
        off = pl.multiple_of(k * 128, 128)
        pltpu.async_copy(wt_hbm.at[:, pl.ds(off, 128)], tiles_v.at[p], l_sems[p])

    def wait_load(p):
        pltpu.make_async_copy(
            wt_hbm.at[:, pl.ds(0, 128)], tiles_v.at[p], l_sems[p]
        ).wait()

    def start_store(p, k):
        off = pl.multiple_of(k * 128, 128)
        pltpu.async_copy(outb_v.at[p], w2x_hbm.at[pl.ds(off, 128)], s_sems[p])

    def wait_store(p):
        pltpu.make_async_copy(
            outb_v.at[p], w2x_hbm.at[pl.ds(0, 128)], s_sems[p]
        ).wait()

    start_load(0, base_k)
    start_load(1, base_k + 1)

    @pl.loop(0, K_PER_W, step=2)
    def _(t):
        for p in range(2):
            k = base_k + t + p
            wait_load(p)

            @pl.when(t + p >= 2)
            def _():
                wait_store(p)

            transpose_tile(tiles_v.at[p], outb_v.at[p])
            start_store(p, k)
            nxt = jnp.minimum(k + 2, base_k + K_PER_W - 1)
            start_load(p, nxt)

    for p in range(2):
        wait_load(p)
        wait_store(p)

    @pl.when(wid < K_REM)
    def _():
        k = KCOLS_FULL - K_REM + wid
        off = k * 128
        pltpu.sync_copy(wt_hbm.at[:, pl.ds(off, 128)], tiles_v.at[0])
        transpose_tile(tiles_v.at[0], outb_v.at[0])
        pltpu.sync_copy(outb_v.at[0], w2x_hbm.at[pl.ds(off, 128)])

    @pl.when(wid == K_REM)
    def _():
        pltpu.sync_copy(wtail_hbm, tail_v)
        for r in range(TAIL_ROWS):
            for g in range(4):
                outb_v[1, r, pl.ds(16 * g, 16)] = tail_v[r, pl.ds(16 * g, 16)]
        pltpu.sync_copy(
            outb_v.at[1, pl.ds(0, TAIL_ROWS)],
            w2x_hbm.at[pl.ds(NUM_EMB - TAIL_ROWS, TAIL_ROWS)],
        )


def _gather_body(idx_hbm, w2x_hbm, out_hbm, idxb0, idxb1, g0, g1,
                 slab0, slab1, i_sems, g_sems, s_sems):
    idxb = (idxb0, idxb1)
    gbuf = (g0, g1)
    slab = (slab0, slab1)
    wid = lax.axis_index("s") * NC + lax.axis_index("c")
    base_b = wid * B_PER_W
    base_l = base_b * HIST

    def start_idx(p, c):
        off = pl.multiple_of(base_l + c * CHUNK, 8)
        pltpu.async_copy(idx_hbm.at[pl.ds(off, CWIN)], idxb[p], i_sems[p])

    def wait_idx(p):
        pltpu.make_async_copy(
            idx_hbm.at[pl.ds(0, CWIN)], idxb[p], i_sems[p]
        ).wait()

    def start_gather(p):
        pltpu.async_copy(w2x_hbm.at[idxb[p]], gbuf[p], g_sems[p])

    def wait_gather(p):
        pltpu.make_async_copy(
            w2x_hbm.at[idxb[p]], gbuf[p], g_sems[p]
        ).wait()

    def start_store(p, c):
        b0 = pl.multiple_of(base_b + c * NB, NB)
        pltpu.async_copy(slab[p], out_hbm.at[pl.ds(b0, NB)], s_sems[p])

    def wait_store(p):
        pltpu.make_async_copy(
            slab[p], out_hbm.at[pl.ds(0, NB)], s_sems[p]
        ).wait()

    def extract(p):
        for j in range(CHUNK):
            b_l, h = j // HIST, j % HIST
            for g in range(4):
                slab[p][b_l, h, pl.ds(16 * g, 16)] = gbuf[p][j, pl.ds(16 * g, 16)]

    start_idx(0, 0)
    start_idx(1, 1)
    wait_idx(0)
    start_gather(0)

    @pl.loop(0, N_CHUNKS, step=2)
    def _(c0):
        for p in range(2):
            c = c0 + p
            wait_gather(p)
            start_idx(p, jnp.minimum(c + 2, N_CHUNKS - 1))
            wait_idx(1 - p)
            start_gather(1 - p)

            @pl.when(c >= 2)
            def _():
                wait_store(p)

            extract(p)
            start_store(p, c)

    wait_gather(0)
    wait_idx(1)
    for p in range(2):
        wait_store(p)


@jax.jit
def _embed(token_ids, W):
    idx = jnp.pad(token_ids.reshape(B), (0, CWIN - CHUNK))
    wt = W.T
    mesh = plsc.VectorSubcoreMesh(core_axis_name="c", subcore_axis_name="s")

    wtail = W[NUM_EMB - TAIL_ROWS:, :]
    w2x = pl.kernel(
        _transpose_body,
        out_type=jax.ShapeDtypeStruct((NUM_EMB, 128), jnp.float32),
        mesh=mesh,
        scratch_types=[
            pltpu.VMEM((2, 64, 128), jnp.float32),
            pltpu.VMEM((2, 128, 128), jnp.float32),
            pltpu.VMEM((TAIL_ROWS, DIM), jnp.float32),
            [pltpu.SemaphoreType.DMA] * 2,
            [pltpu.SemaphoreType.DMA] * 2,
        ],
        compiler_params=pltpu.CompilerParams(needs_layout_passes=False),
    )(wt, wtail)

    out = pl.kernel(
        _gather_body,
        out_type=jax.ShapeDtypeStruct((BATCH, HIST, DIM), jnp.float32),
        mesh=mesh,
        scratch_types=[
            pltpu.VMEM((CWIN,), jnp.int32),
            pltpu.VMEM((CWIN,), jnp.int32),
            pltpu.VMEM((CWIN, 128), jnp.float32),
            pltpu.VMEM((CWIN, 128), jnp.float32),
            pltpu.VMEM((NB, HIST, DIM), jnp.float32),
            pltpu.VMEM((NB, HIST, DIM), jnp.float32),
            [pltpu.SemaphoreType.DMA] * 2,
            [pltpu.SemaphoreType.DMA] * 2,
            [pltpu.SemaphoreType.DMA] * 2,
        ],
    )(idx, w2x)
    return out


def kernel(token_ids, W):
    return _embed(token_ids, W)


# speedup vs baseline: 1.9900x; 1.0441x over previous
import jax
import jax.numpy as jnp
from jax import lax
from jax.experimental import pallas as pl
from jax.experimental.pallas import tpu as pltpu
from jax.experimental.pallas import tpu_sc as plsc

NUM_EMB = 1000000
DIM = 64
BATCH = 16384
HIST = 50

NC = 2
NS = 16
NW = NC * NS

B = BATCH * HIST

KCOLS_FULL = NUM_EMB // 128
K_PER_W = KCOLS_FULL // NW
K_REM = KCOLS_FULL - K_PER_W * NW
TAIL_ROWS = NUM_EMB - KCOLS_FULL * 128

B_PER_W = BATCH // NW
NB = 4
CHUNK = NB * HIST
CWIN = 256
N_CHUNKS = B_PER_W // NB
assert CHUNK % 8 == 0 and N_CHUNKS % 2 == 0

_IOTA16 = None


def _transpose_body(wt_hbm, wtail_hbm, w2x_hbm, tiles_v, outb_v, tail_v,
                    l_sems, s_sems):
    wid = lax.axis_index("s") * NC + lax.axis_index("c")
    base_k = wid * K_PER_W

    iota16 = lax.broadcasted_iota(jnp.int32, (16,), 0)

    perms = [(iota16 + j) & 15 for j in range(16)]

    def transpose_tile(tv, ov):
        @pl.loop(0, 32)
        def _(t):
            d0 = (t // 8) * 16
            l0 = (t % 8) * 16
            dvec = iota16 + d0
            for j in range(16):
                lvec = perms[j] + l0
                vals = plsc.load_gather(tv, [dvec, lvec])
                plsc.store_scatter(ov, [lvec, dvec], vals)

    def start_load(p, k):
        off = pl.multiple_of(k * 128, 128)
        pltpu.async_copy(wt_hbm.at[:, pl.ds(off, 128)], tiles_v.at[p], l_sems[p])

    def wait_load(p):
        pltpu.make_async_copy(
            wt_hbm.at[:, pl.ds(0, 128)], tiles_v.at[p], l_sems[p]
        ).wait()

    def start_store(p, k):
        off = pl.multiple_of(k * 128, 128)
        pltpu.async_copy(outb_v.at[p], w2x_hbm.at[pl.ds(off, 128)], s_sems[p])

    def wait_store(p):
        pltpu.make_async_copy(
            outb_v.at[p], w2x_hbm.at[pl.ds(0, 128)], s_sems[p]
        ).wait()

    start_load(0, base_k)
    start_load(1, base_k + 1)

    @pl.loop(0, K_PER_W, step=2)
    def _(t):
        for p in range(2):
            k = base_k + t + p
            wait_load(p)

            @pl.when(t + p >= 2)
            def _():
                wait_store(p)

            transpose_tile(tiles_v.at[p], outb_v.at[p])
            start_store(p, k)
            nxt = jnp.minimum(k + 2, base_k + K_PER_W - 1)
            start_load(p, nxt)

    for p in range(2):
        wait_load(p)
        wait_store(p)

    @pl.when(wid < K_REM)
    def _():
        k = KCOLS_FULL - K_REM + wid
        off = k * 128
        pltpu.sync_copy(wt_hbm.at[:, pl.ds(off, 128)], tiles_v.at[0])
        transpose_tile(tiles_v.at[0], outb_v.at[0])
        pltpu.sync_copy(outb_v.at[0], w2x_hbm.at[pl.ds(off, 128)])

    @pl.when(wid == K_REM)
    def _():
        pltpu.sync_copy(wtail_hbm, tail_v)
        for r in range(TAIL_ROWS):
            for g in range(4):
                outb_v[1, r, pl.ds(16 * g, 16)] = tail_v[r, pl.ds(16 * g, 16)]
        pltpu.sync_copy(
            outb_v.at[1, pl.ds(0, TAIL_ROWS)],
            w2x_hbm.at[pl.ds(NUM_EMB - TAIL_ROWS, TAIL_ROWS)],
        )


def _gather_body(idx_hbm, w2x_hbm, out_hbm, idxb0, idxb1, g0, g1,
                 slab0, slab1, i_sems, g_sems, s_sems):
    idxb = (idxb0, idxb1)
    gbuf = (g0, g1)
    slab = (slab0, slab1)
    wid = lax.axis_index("s") * NC + lax.axis_index("c")
    base_b = wid * B_PER_W
    base_l = base_b * HIST

    def start_idx(p, c):
        off = pl.multiple_of(base_l + c * CHUNK, 8)
        pltpu.async_copy(idx_hbm.at[pl.ds(off, CWIN)], idxb[p], i_sems[p])

    def wait_idx(p):
        pltpu.make_async_copy(
            idx_hbm.at[pl.ds(0, CWIN)], idxb[p], i_sems[p]
        ).wait()

    def start_gather(p):
        pltpu.async_copy(
            w2x_hbm.at[idxb[p].at[pl.ds(0, CHUNK)]], gbuf[p], g_sems[p]
        )

    def wait_gather(p):
        pltpu.make_async_copy(
            w2x_hbm.at[idxb[p].at[pl.ds(0, CHUNK)]], gbuf[p], g_sems[p]
        ).wait()

    def start_store(p, c):
        b0 = pl.multiple_of(base_b + c * NB, NB)
        pltpu.async_copy(slab[p], out_hbm.at[pl.ds(b0, NB)], s_sems[p])

    def wait_store(p):
        pltpu.make_async_copy(
            slab[p], out_hbm.at[pl.ds(0, NB)], s_sems[p]
        ).wait()

    def extract(p):
        for j in range(CHUNK):
            b_l, h = j // HIST, j % HIST
            for g in range(4):
                slab[p][b_l, h, pl.ds(16 * g, 16)] = gbuf[p][j, pl.ds(16 * g, 16)]

    start_idx(0, 0)
    start_idx(1, 1)
    wait_idx(0)
    start_gather(0)

    @pl.loop(0, N_CHUNKS, step=2)
    def _(c0):
        for p in range(2):
            c = c0 + p
            wait_gather(p)
            start_idx(p, jnp.minimum(c + 2, N_CHUNKS - 1))
            wait_idx(1 - p)
            start_gather(1 - p)

            @pl.when(c >= 2)
            def _():
                wait_store(p)

            extract(p)
            start_store(p, c)

    wait_gather(0)
    wait_idx(1)
    for p in range(2):
        wait_store(p)


@jax.jit
def _embed(token_ids, W):
    idx = jnp.pad(token_ids.reshape(B), (0, CWIN - CHUNK))
    wt = W.T
    mesh = plsc.VectorSubcoreMesh(core_axis_name="c", subcore_axis_name="s")

    wtail = W[NUM_EMB - TAIL_ROWS:, :]
    w2x = pl.kernel(
        _transpose_body,
        out_type=jax.ShapeDtypeStruct((NUM_EMB, 128), jnp.float32),
        mesh=mesh,
        scratch_types=[
            pltpu.VMEM((2, 64, 128), jnp.float32),
            pltpu.VMEM((2, 128, 128), jnp.float32),
            pltpu.VMEM((TAIL_ROWS, DIM), jnp.float32),
            [pltpu.SemaphoreType.DMA] * 2,
            [pltpu.SemaphoreType.DMA] * 2,
        ],
        compiler_params=pltpu.CompilerParams(needs_layout_passes=False),
    )(wt, wtail)

    out = pl.kernel(
        _gather_body,
        out_type=jax.ShapeDtypeStruct((BATCH, HIST, DIM), jnp.float32),
        mesh=mesh,
        scratch_types=[
            pltpu.VMEM((CWIN,), jnp.int32),
            pltpu.VMEM((CWIN,), jnp.int32),
            pltpu.VMEM((CHUNK, 128), jnp.float32),
            pltpu.VMEM((CHUNK, 128), jnp.float32),
            pltpu.VMEM((NB, HIST, DIM), jnp.float32),
            pltpu.VMEM((NB, HIST, DIM), jnp.float32),
            [pltpu.SemaphoreType.DMA] * 2,
            [pltpu.SemaphoreType.DMA] * 2,
            [pltpu.SemaphoreType.DMA] * 2,
        ],
    )(idx, w2x)
    return out


def kernel(token_ids, W):
    return _embed(token_ids, W)


# speedup vs baseline: 2.4091x; 1.2106x over previous
import jax
import jax.numpy as jnp
from jax import lax
from jax.experimental import pallas as pl
from jax.experimental.pallas import tpu as pltpu
from jax.experimental.pallas import tpu_sc as plsc

NUM_EMB = 1000000
DIM = 64
BATCH = 16384
HIST = 50

NC = 2
NS = 16
NW = NC * NS

B = BATCH * HIST

KCOLS_FULL = NUM_EMB // 128
K_PER_W = KCOLS_FULL // NW
K_REM = KCOLS_FULL - K_PER_W * NW
TAIL_ROWS = NUM_EMB - KCOLS_FULL * 128

B_PER_W = BATCH // NW
NB = 4
CHUNK = NB * HIST
CWIN = 256
N_CHUNKS = B_PER_W // NB
assert CHUNK % 8 == 0 and N_CHUNKS % 2 == 0

_IOTA16 = None


def _transpose_body(wt_hbm, wtail_hbm, w2x_hbm, tiles_v, outb_v, tail_v,
                    l_sems, s_sems):
    wid = lax.axis_index("s") * NC + lax.axis_index("c")
    base_k = wid * K_PER_W

    iota16 = lax.broadcasted_iota(jnp.int32, (16,), 0)

    perms = [(iota16 + j) & 15 for j in range(16)]

    def transpose_tile(tv, ov):
        @pl.loop(0, 32)
        def _(t):
            d0 = (t // 8) * 16
            l0 = (t % 8) * 16
            dvec = iota16 + d0
            for j in range(16):
                lvec = perms[j] + l0
                vals = plsc.load_gather(tv, [dvec, lvec])
                plsc.store_scatter(ov, [lvec, dvec], vals)

    def start_load(p, k):
        off = pl.multiple_of(k * 128, 128)
        pltpu.async_copy(wt_hbm.at[:, pl.ds(off, 128)], tiles_v.at[p], l_sems[p])

    def wait_load(p):
        pltpu.make_async_copy(
            wt_hbm.at[:, pl.ds(0, 128)], tiles_v.at[p], l_sems[p]
        ).wait()

    def start_store(p, k):
        off = pl.multiple_of(k * 128, 128)
        pltpu.async_copy(outb_v.at[p], w2x_hbm.at[pl.ds(off, 128)], s_sems[p])

    def wait_store(p):
        pltpu.make_async_copy(
            outb_v.at[p], w2x_hbm.at[pl.ds(0, 128)], s_sems[p]
        ).wait()

    start_load(0, base_k)
    start_load(1, base_k + 1)

    @pl.loop(0, K_PER_W, step=2)
    def _(t):
        for p in range(2):
            k = base_k + t + p
            wait_load(p)

            @pl.when(t + p >= 2)
            def _():
                wait_store(p)

            transpose_tile(tiles_v.at[p], outb_v.at[p])
            start_store(p, k)
            nxt = jnp.minimum(k + 2, base_k + K_PER_W - 1)
            start_load(p, nxt)

    for p in range(2):
        wait_load(p)
        wait_store(p)

    @pl.when(wid < K_REM)
    def _():
        k = KCOLS_FULL - K_REM + wid
        off = k * 128
        pltpu.sync_copy(wt_hbm.at[:, pl.ds(off, 128)], tiles_v.at[0])
        transpose_tile(tiles_v.at[0], outb_v.at[0])
        pltpu.sync_copy(outb_v.at[0], w2x_hbm.at[pl.ds(off, 128)])

    @pl.when(wid == K_REM)
    def _():
        pltpu.sync_copy(wtail_hbm, tail_v)
        for r in range(TAIL_ROWS):
            for g in range(4):
                outb_v[1, r, pl.ds(16 * g, 16)] = tail_v[r, pl.ds(16 * g, 16)]
        pltpu.sync_copy(
            outb_v.at[1, pl.ds(0, TAIL_ROWS)],
            w2x_hbm.at[pl.ds(NUM_EMB - TAIL_ROWS, TAIL_ROWS)],
        )


NBLK = HIST * (BATCH // 128)
BLK_PER_W = NBLK // NW
KB = BATCH // 128


def _gather_body(idx_hbm, w2x_hbm, out_hbm, idxb0, idxb1, g0, g1,
                 o0, o1, i_sems, g_sems, s_sems):
    idxb = (idxb0, idxb1)
    gbuf = (g0, g1)
    obuf = (o0, o1)
    wid = lax.axis_index("s") * NC + lax.axis_index("c")
    base_m = wid * BLK_PER_W

    iota16 = lax.broadcasted_iota(jnp.int32, (16,), 0)
    perms = [(iota16 + j) & 15 for j in range(16)]

    def start_idx(p, m):
        off = pl.multiple_of(m * 128, 128)
        pltpu.async_copy(idx_hbm.at[pl.ds(off, 128)], idxb[p], i_sems[p])

    def wait_idx(p):
        pltpu.make_async_copy(
            idx_hbm.at[pl.ds(0, 128)], idxb[p], i_sems[p]
        ).wait()

    def start_gather(p):
        pltpu.async_copy(w2x_hbm.at[idxb[p]], gbuf[p], g_sems[p])

    def wait_gather(p):
        pltpu.make_async_copy(
            w2x_hbm.at[idxb[p]], gbuf[p], g_sems[p]
        ).wait()

    def start_store(p, m):
        h = m // KB
        k = m % KB
        off = pl.multiple_of(k * 128, 128)
        pltpu.async_copy(
            obuf[p], out_hbm.at[h, :, pl.ds(off, 128)], s_sems[p]
        )

    def wait_store(p):
        pltpu.make_async_copy(
            obuf[p], out_hbm.at[0, :, pl.ds(0, 128)], s_sems[p]
        ).wait()

    def transpose_block(p):
        gv, ov = gbuf[p], obuf[p]

        @pl.loop(0, 32)
        def _(t):
            d0 = (t // 8) * 16
            b0 = (t % 8) * 16
            dvec = iota16 + d0
            for j in range(16):
                bvec = perms[j] + b0
                vals = plsc.load_gather(gv, [bvec, dvec])
                plsc.store_scatter(ov, [dvec, bvec], vals)

    start_idx(0, base_m)
    start_idx(1, base_m + 1)
    wait_idx(0)
    start_gather(0)

    @pl.loop(0, BLK_PER_W, step=2)
    def _(c0):
        for p in range(2):
            m = base_m + c0 + p
            wait_gather(p)
            start_idx(p, jnp.minimum(m + 2, base_m + BLK_PER_W - 1))
            wait_idx(1 - p)
            start_gather(1 - p)

            @pl.when(c0 + p >= 2)
            def _():
                wait_store(p)

            transpose_block(p)
            start_store(p, m)

    wait_gather(0)
    wait_idx(1)
    for p in range(2):
        wait_store(p)


@jax.jit
def _embed(token_ids, W):
    idx = token_ids.T.reshape(B)
    wt = W.T
    mesh = plsc.VectorSubcoreMesh(core_axis_name="c", subcore_axis_name="s")

    wtail = W[NUM_EMB - TAIL_ROWS:, :]
    w2x = pl.kernel(
        _transpose_body,
        out_type=jax.ShapeDtypeStruct((NUM_EMB, 128), jnp.float32),
        mesh=mesh,
        scratch_types=[
            pltpu.VMEM((2, 64, 128), jnp.float32),
            pltpu.VMEM((2, 128, 128), jnp.float32),
            pltpu.VMEM((TAIL_ROWS, DIM), jnp.float32),
            [pltpu.SemaphoreType.DMA] * 2,
            [pltpu.SemaphoreType.DMA] * 2,
        ],
        compiler_params=pltpu.CompilerParams(needs_layout_passes=False),
    )(wt, wtail)

    outp = pl.kernel(
        _gather_body,
        out_type=jax.ShapeDtypeStruct((HIST, DIM, BATCH), jnp.float32),
        mesh=mesh,
        scratch_types=[
            pltpu.VMEM((128,), jnp.int32),
            pltpu.VMEM((128,), jnp.int32),
            pltpu.VMEM((128, 128), jnp.float32),
            pltpu.VMEM((128, 128), jnp.float32),
            pltpu.VMEM((DIM, 128), jnp.float32),
            pltpu.VMEM((DIM, 128), jnp.float32),
            [pltpu.SemaphoreType.DMA] * 2,
            [pltpu.SemaphoreType.DMA] * 2,
            [pltpu.SemaphoreType.DMA] * 2,
        ],
        compiler_params=pltpu.CompilerParams(needs_layout_passes=False),
    )(idx, w2x)
    return jnp.transpose(outp, (2, 0, 1))


def kernel(token_ids, W):
    return _embed(token_ids, W)
